# baseline (speedup 1.0000x reference)
import functools

import jax
import jax.numpy as jnp
from jax import lax
from jax.experimental import pallas as pl
from jax.experimental.pallas import tpu as pltpu
from jax.experimental.pallas import tpu_sc as plsc

B = 16384
EMB = 256
UDIM = 64
E = 16
C = 100
CP = 128
IND = 80
HID = 64
NPAIR = 2 * B
T = 256
P = NPAIR + E * T
NT = P // T
NC = 2
NS = 16
NW = NC * NS

_SC_MESH = dict(core_axis_name="c", subcore_axis_name="s")


def _wid():
    return lax.axis_index("s") * NC + lax.axis_index("c")


def _sc_gather_u(table, idx):
    per = B // NW

    @functools.partial(
        pl.kernel,
        out_type=jax.ShapeDtypeStruct((B, 128), jnp.float32),
        mesh=plsc.VectorSubcoreMesh(**_SC_MESH),
        scratch_types=[
            pltpu.VMEM((per,), jnp.int32),
            pltpu.VMEM((per, 128), jnp.float32),
            pltpu.SemaphoreType.DMA,
        ],
    )
    def k(table_hbm, idx_hbm, out_hbm, idx_v, rows_v, sem):
        base = _wid() * per
        pltpu.sync_copy(idx_hbm.at[pl.ds(base, per)], idx_v)
        pltpu.async_copy(table_hbm.at[idx_v], rows_v, sem).wait()
        pltpu.sync_copy(rows_v, out_hbm.at[pl.ds(base, per)])

    return k(table, idx)


def _gelu(z):
    return 0.5 * z * (1.0 + lax.erf(z * 0.7071067811865476))


_HI_MASK = -65536


def _pack_bf16_pair(a, b):
    ab = lax.bitcast_convert_type(
        a.astype(jnp.bfloat16).astype(jnp.float32), jnp.int32)
    bb = lax.bitcast_convert_type(
        b.astype(jnp.bfloat16).astype(jnp.float32), jnp.int32)
    return jnp.bitwise_or(
        jnp.bitwise_and(bb, jnp.full(bb.shape, _HI_MASK, jnp.int32)),
        lax.shift_right_logical(ab, 16))


def _unpack_lo(v):
    return lax.bitcast_convert_type(
        lax.shift_left(v, 16), jnp.float32).astype(jnp.bfloat16)


def _unpack_hi(v):
    return lax.bitcast_convert_type(
        jnp.bitwise_and(v, jnp.full(v.shape, _HI_MASK, jnp.int32)),
        jnp.float32).astype(jnp.bfloat16)


def _ln(z, g, beta):
    mu = jnp.mean(z, axis=-1, keepdims=True)
    d = z - mu
    var = jnp.mean(d * d, axis=-1, keepdims=True)
    return d * lax.rsqrt(var + 1e-5) * g + beta


def _bb_body(xf_ref, w1_ref, b1_ref, w2_ref, b2_ref, g_ref, beta_ref,
             h32_ref, hf_ref):
    xf = xf_ref[...]
    h1 = _gelu(jnp.dot(xf, w1_ref[...], preferred_element_type=jnp.float32)
               + b1_ref[...])
    h2 = _gelu(jnp.dot(h1, w2_ref[...], preferred_element_type=jnp.float32)
               + b2_ref[...])
    h = _ln(h2, g_ref[...], beta_ref[...])
    h32_ref[...] = _pack_bf16_pair(h[:, :128], h[:, 128:])
    hf_ref[...] = h


def _backbone(xf, w1, b1, w2, b2, g, beta):
    BT = 1024
    grid = B // BT
    return pl.pallas_call(
        _bb_body,
        grid=(grid,),
        in_specs=[
            pl.BlockSpec((BT, IND), lambda i: (i, 0)),
            pl.BlockSpec((IND, HID), lambda i: (0, 0)),
            pl.BlockSpec((1, HID), lambda i: (0, 0)),
            pl.BlockSpec((HID, EMB), lambda i: (0, 0)),
            pl.BlockSpec((1, EMB), lambda i: (0, 0)),
            pl.BlockSpec((1, EMB), lambda i: (0, 0)),
            pl.BlockSpec((1, EMB), lambda i: (0, 0)),
        ],
        out_specs=[
            pl.BlockSpec((BT, 128), lambda i: (i, 0)),
            pl.BlockSpec((BT, EMB), lambda i: (i, 0)),
        ],
        out_shape=[
            jax.ShapeDtypeStruct((B, 128), jnp.int32),
            jax.ShapeDtypeStruct((B, EMB), jnp.float32),
        ],
        compiler_params=pltpu.CompilerParams(
            dimension_semantics=("parallel",)),
    )(xf, w1, b1, w2, b2, g, beta)


def _gate_body(h_ref, u_ref, gwh_ref, gwu_ref, gb_ref,
               gi0_ref, gi1_ref, wc_ref):
    h = h_ref[...]
    u = u_ref[...]
    l = (jnp.dot(h, gwh_ref[...], preferred_element_type=jnp.float32)
         + jnp.dot(u, gwu_ref[...], preferred_element_type=jnp.float32)
         + gb_ref[...])
    BT = l.shape[0]
    iota = lax.broadcasted_iota(jnp.int32, (BT, E), 1)
    big = jnp.int32(999)
    m0 = jnp.max(l, axis=1, keepdims=True)
    i0 = jnp.min(jnp.where(l >= m0, iota, big), axis=1, keepdims=True)
    l2 = jnp.where(iota == i0, -1e30, l)
    m1 = jnp.max(l2, axis=1, keepdims=True)
    i1 = jnp.min(jnp.where(l2 >= m1, iota, big), axis=1, keepdims=True)
    w0 = 1.0 / (1.0 + jnp.exp(m1 - m0))
    w1 = 1.0 - w0
    vals = jnp.concatenate(
        [i0.astype(jnp.float32), i1.astype(jnp.float32)], axis=1)
    eye = (lax.broadcasted_iota(jnp.int32, (128, 128), 0)
           == lax.broadcasted_iota(jnp.int32, (128, 128), 1)
           ).astype(jnp.float32)
    r0 = []
    r1 = []
    for cch in range(BT // 128):
        blk = vals[cch * 128:(cch + 1) * 128, :2]
        ck = lax.dot_general(blk, eye, (((0,), (0,)), ((), ())),
                             preferred_element_type=jnp.float32)
        r0.append(ck[0:1])
        r1.append(ck[1:2])
    gi0_ref[...] = jnp.concatenate(r0, axis=0)
    gi1_ref[...] = jnp.concatenate(r1, axis=0)
    wc_ref[...] = jnp.concatenate(
        [jnp.broadcast_to(w0, (BT, 16)), jnp.broadcast_to(w1, (BT, 16)),
         jnp.zeros((BT, 96), jnp.float32)], axis=1)


def _gate(hf, u, gwh, gwu, gb):
    BT = 1024
    grid = B // BT
    nr = BT // 128
    return pl.pallas_call(
        _gate_body,
        grid=(grid,),
        in_specs=[
            pl.BlockSpec((BT, EMB), lambda i: (i, 0)),
            pl.BlockSpec((BT, 128), lambda i: (i, 0)),
            pl.BlockSpec((EMB, E), lambda i: (0, 0)),
            pl.BlockSpec((128, E), lambda i: (0, 0)),
            pl.BlockSpec((1, E), lambda i: (0, 0)),
        ],
        out_specs=[
            pl.BlockSpec((nr, 128), lambda i: (i, 0)),
            pl.BlockSpec((nr, 128), lambda i: (i, 0)),
            pl.BlockSpec((BT, 128), lambda i: (i, 0)),
        ],
        out_shape=[
            jax.ShapeDtypeStruct((B // 128, 128), jnp.float32),
            jax.ShapeDtypeStruct((B // 128, 128), jnp.float32),
            jax.ShapeDtypeStruct((B, 128), jnp.float32),
        ],
        compiler_params=pltpu.CompilerParams(
            dimension_semantics=("parallel",)),
    )(hf, u, gwh, gwu, gb)


def _route_body(gi0_ref, gi1_ref, pos_ref, te_ref):
    ep = jnp.concatenate([gi0_ref[...], gi1_ref[...]], axis=0).astype(jnp.int32)
    nrow = ep.shape[0]
    f32 = jnp.float32
    r128 = lax.broadcasted_iota(jnp.int32, (128, 128), 0)
    c128 = lax.broadcasted_iota(jnp.int32, (128, 128), 1)
    su128 = (r128 < c128).astype(f32)
    rr = lax.broadcasted_iota(jnp.int32, (nrow, nrow), 0)
    cc = lax.broadcasted_iota(jnp.int32, (nrow, nrow), 1)
    slr = (cc < rr).astype(f32)
    masks = []
    withins = []
    rowsums = []
    for e in range(E):
        me = (ep == e).astype(f32)
        masks.append(me)
        withins.append(jnp.dot(me, su128, preferred_element_type=f32))
        rowsums.append(jnp.sum(me, axis=1, keepdims=True))
    rs = jnp.concatenate(rowsums, axis=1)
    rowbase = jnp.dot(slr, rs, preferred_element_type=f32)
    ones_col = jnp.ones((nrow, 1), f32)
    counts = lax.dot_general(rs, ones_col, (((0,), (0,)), ((), ())),
                             preferred_element_type=f32)
    ci = counts.astype(jnp.int32)
    padded = ((ci + (T - 1)) // T) * T
    r16 = lax.broadcasted_iota(jnp.int32, (E, E), 0)
    c16 = lax.broadcasted_iota(jnp.int32, (E, E), 1)
    sl16 = (c16 < r16).astype(f32)
    qcol = jnp.dot(sl16, padded.astype(f32),
                   preferred_element_type=f32)
    acc = jnp.zeros_like(masks[0])
    for e in range(E):
        base_e = qcol[e:e + 1, 0:1] + rowbase[:, e:e + 1]
        acc = acc + masks[e] * (base_e + withins[e])
    pos_ref[...] = acc.astype(jnp.int32)
    jt = lax.broadcasted_iota(jnp.int32, (E, NT), 1) * T
    cmp = (qcol.astype(jnp.int32) <= jt).astype(jnp.int32)
    te = jnp.sum(cmp, axis=0, keepdims=True) - 1
    te_ref[...] = jnp.broadcast_to(te, (8, NT))


def _route(gi0, gi1):
    nrow = NPAIR // 128
    return pl.pallas_call(
        _route_body,
        in_specs=[
            pl.BlockSpec((B // 128, 128), lambda: (0, 0)),
            pl.BlockSpec((B // 128, 128), lambda: (0, 0)),
        ],
        out_specs=[
            pl.BlockSpec((nrow, 128), lambda: (0, 0)),
            pl.BlockSpec((8, NT), lambda: (0, 0)),
        ],
        out_shape=[
            jax.ShapeDtypeStruct((nrow, 128), jnp.int32),
            jax.ShapeDtypeStruct((8, NT), jnp.int32),
        ],
    )(gi0, gi1)


def _sc_dispatch(h, pos):
    per = NPAIR // NW
    SUB = 128
    nsub = per // SUB

    @functools.partial(
        pl.kernel,
        out_type=jax.ShapeDtypeStruct((P, 128), jnp.int32),
        mesh=plsc.VectorSubcoreMesh(**_SC_MESH),
        scratch_types=[
            pltpu.VMEM((SUB,), jnp.int32),
            pltpu.VMEM((SUB,), jnp.int32),
            pltpu.VMEM((SUB,), jnp.int32),
            pltpu.VMEM((SUB,), jnp.int32),
            pltpu.VMEM((SUB, 128), jnp.int32),
            pltpu.VMEM((SUB, 128), jnp.int32),
            pltpu.SemaphoreType.DMA,
            pltpu.SemaphoreType.DMA,
            pltpu.SemaphoreType.DMA,
            pltpu.SemaphoreType.DMA,
        ],
    )
    def k(h_hbm, pos_hbm, hs_hbm, tok0, tok1, pos0, pos1, r0, r1,
          gs0, gs1, ss0, ss1):
        toks = [tok0, tok1]
        poss = [pos0, pos1]
        rows = [r0, r1]
        gss = [gs0, gs1]
        sss = [ss0, ss1]
        gh = [None, None]
        sh = [None, None]
        w = _wid()
        for s in range(nsub):
            b = s % 2
            if s >= 2:
                sh[b].wait()
            base = w * per + s * SUB
            bm = base % B

            def fill(c, _):
                toks[b][pl.ds(c * 16, 16)] = (
                    bm + c * 16 + lax.iota(jnp.int32, 16))
                return 0

            lax.fori_loop(0, SUB // 16, fill, 0)
            pltpu.sync_copy(pos_hbm.at[base // 128], poss[b])
            gh[b] = pltpu.async_copy(h_hbm.at[toks[b]], rows[b], gss[b])
            if s >= 1:
                c = (s - 1) % 2
                gh[c].wait()
                sh[c] = pltpu.async_copy(rows[c], hs_hbm.at[poss[c]], sss[c])
        last = (nsub - 1) % 2
        gh[last].wait()
        sh[last] = pltpu.async_copy(rows[last], hs_hbm.at[poss[last]],
                                    sss[last])
        if nsub >= 2:
            sh[(nsub - 2) % 2].wait()
        sh[last].wait()

    return k(h, pos)


def _expert_body(te_ref, hs_ref, w1_ref, b1_ref, g_ref, beta_ref,
                 w2_ref, b2_ref, ys_ref):
    hs32 = hs_ref[...]
    w1 = w1_ref[0]
    z = (jnp.dot(_unpack_lo(hs32), w1[:128],
                 preferred_element_type=jnp.float32)
         + jnp.dot(_unpack_hi(hs32), w1[128:],
                   preferred_element_type=jnp.float32)) + b1_ref[0]
    z = _gelu(z)
    z = _ln(z, g_ref[0], beta_ref[0])
    y = jnp.dot(z.astype(jnp.bfloat16), w2_ref[0],
                preferred_element_type=jnp.float32) + b2_ref[0]
    ys_ref[...] = y


def _experts(te, hs, w1, b1, g, beta, w2p, b2p):
    grid_spec = pltpu.PrefetchScalarGridSpec(
        num_scalar_prefetch=1,
        grid=(NT,),
        in_specs=[
            pl.BlockSpec((T, 128), lambda i, te: (i, 0)),
            pl.BlockSpec((1, EMB, EMB), lambda i, te: (te[i], 0, 0)),
            pl.BlockSpec((1, 1, EMB), lambda i, te: (te[i], 0, 0)),
            pl.BlockSpec((1, 1, EMB), lambda i, te: (te[i], 0, 0)),
            pl.BlockSpec((1, 1, EMB), lambda i, te: (te[i], 0, 0)),
            pl.BlockSpec((1, EMB, CP), lambda i, te: (te[i], 0, 0)),
            pl.BlockSpec((1, 1, CP), lambda i, te: (te[i], 0, 0)),
        ],
        out_specs=pl.BlockSpec((T, CP), lambda i, te: (i, 0)),
    )
    return pl.pallas_call(
        _expert_body,
        grid_spec=grid_spec,
        out_shape=jax.ShapeDtypeStruct((P, CP), jnp.float32),
        compiler_params=pltpu.CompilerParams(
            dimension_semantics=("arbitrary",)),
    )(te, hs, w1, b1, g, beta, w2p, b2p)


def _sc_combine(ys, pos2d, wc):
    per = B // NW
    SUB = 128
    nsub = per // SUB

    @functools.partial(
        pl.kernel,
        out_type=jax.ShapeDtypeStruct((B, C), jnp.float32),
        mesh=plsc.VectorSubcoreMesh(**_SC_MESH),
        scratch_types=[
            pltpu.VMEM((SUB,), jnp.int32),
            pltpu.VMEM((SUB,), jnp.int32),
            pltpu.VMEM((SUB,), jnp.int32),
            pltpu.VMEM((SUB,), jnp.int32),
            pltpu.VMEM((SUB, 128), jnp.float32),
            pltpu.VMEM((SUB, 128), jnp.float32),
            pltpu.VMEM((SUB, CP), jnp.float32),
            pltpu.VMEM((SUB, CP), jnp.float32),
            pltpu.VMEM((SUB, CP), jnp.float32),
            pltpu.VMEM((SUB, CP), jnp.float32),
            pltpu.VMEM((SUB, C), jnp.float32),
            pltpu.SemaphoreType.DMA,
            pltpu.SemaphoreType.DMA,
            pltpu.SemaphoreType.DMA,
            pltpu.SemaphoreType.DMA,
        ],
    )
    def k(ys_hbm, pos_hbm, wc_hbm, out_hbm,
          i0a, i0b, i1a, i1b, wca, wcb, g0a, g0b, g1a, g1b, ob,
          s0a, s0b, s1a, s1b):
        i0s = [i0a, i0b]
        i1s = [i1a, i1b]
        wcs = [wca, wcb]
        g0s = [g0a, g0b]
        g1s = [g1a, g1b]
        sm0 = [s0a, s0b]
        sm1 = [s1a, s1b]
        h0 = [None, None]
        h1 = [None, None]
        w = _wid()

        def start(s):
            b = s % 2
            base = w * per + s * SUB
            row = base // 128
            pltpu.sync_copy(pos_hbm.at[row], i0s[b])
            pltpu.sync_copy(pos_hbm.at[(B // 128) + row], i1s[b])
            pltpu.sync_copy(wc_hbm.at[pl.ds(base, SUB)], wcs[b])
            h0[b] = pltpu.async_copy(ys_hbm.at[i0s[b]], g0s[b], sm0[b])
            h1[b] = pltpu.async_copy(ys_hbm.at[i1s[b]], g1s[b], sm1[b])

        start(0)
        for s in range(nsub):
            b = s % 2
            if s + 1 < nsub:
                start(s + 1)
            h0[b].wait()
            h1[b].wait()
            g0 = g0s[b]
            g1 = g1s[b]
            wc_v = wcs[b]

            def body(r, _):
                a = wc_v[r, pl.ds(0, 16)]
                c = wc_v[r, pl.ds(16, 16)]
                for off in (0, 16, 32, 48, 64, 80, C - 16):
                    sl = pl.ds(off, 16)
                    ob[r, sl] = a * g0[r, sl] + c * g1[r, sl]
                return 0

            lax.fori_loop(0, SUB, body, 0)
            base = w * per + s * SUB
            pltpu.sync_copy(ob, out_hbm.at[pl.ds(base, SUB)])

    return k(ys, pos2d, wc)


def kernel(x, user_ids, bb_W1, bb_b1, bb_W2, bb_b2, bb_g, bb_beta,
           user_table, gate_W, gate_b,
           ex_W1, ex_b1, ex_g, ex_beta, ex_W2, ex_b2):
    i32 = jnp.int32
    xf = x.reshape(B, IND)
    table_p = jnp.pad(user_table, ((0, 0), (0, 128 - UDIM)))
    u = _sc_gather_u(table_p, user_ids.astype(i32))
    h, hf = _backbone(xf, bb_W1, bb_b1.reshape(1, HID), bb_W2,
                      bb_b2.reshape(1, EMB), bb_g.reshape(1, EMB),
                      bb_beta.reshape(1, EMB))
    gwu_p = jnp.pad(gate_W[EMB:], ((0, 128 - UDIM), (0, 0)))
    gi0, gi1, wc = _gate(hf, u, gate_W[:EMB], gwu_p, gate_b.reshape(1, E))
    pos2d, te8 = _route(gi0, gi1)
    te = te8[0]
    hs = _sc_dispatch(h, pos2d)
    w2p = jnp.pad(ex_W2, ((0, 0), (0, 0), (0, CP - C))).astype(jnp.bfloat16)
    b2p = jnp.pad(ex_b2, ((0, 0), (0, CP - C))).reshape(E, 1, CP)
    ys = _experts(te, hs, ex_W1.astype(jnp.bfloat16), ex_b1.reshape(E, 1, EMB),
                  ex_g.reshape(E, 1, EMB), ex_beta.reshape(E, 1, EMB),
                  w2p, b2p)
    return _sc_combine(ys, pos2d, wc)


# speedup vs baseline: 1.3024x; 1.3024x over previous
import functools

import jax
import jax.numpy as jnp
from jax import lax
from jax.experimental import pallas as pl
from jax.experimental.pallas import tpu as pltpu
from jax.experimental.pallas import tpu_sc as plsc

B = 16384
EMB = 256
UDIM = 64
E = 16
C = 100
CP = 128
IND = 80
HID = 64
NPAIR = 2 * B
T = 1024
P = NPAIR + E * T
NT = P // T
NC = 2
NS = 16
NW = NC * NS

_SC_MESH = dict(core_axis_name="c", subcore_axis_name="s")


def _wid():
    return lax.axis_index("s") * NC + lax.axis_index("c")


def _sc_gather_u(table, idx):
    per = B // NW

    @functools.partial(
        pl.kernel,
        out_type=jax.ShapeDtypeStruct((B, 128), jnp.float32),
        mesh=plsc.VectorSubcoreMesh(**_SC_MESH),
        scratch_types=[
            pltpu.VMEM((per,), jnp.int32),
            pltpu.VMEM((per, 128), jnp.float32),
            pltpu.SemaphoreType.DMA,
        ],
    )
    def k(table_hbm, idx_hbm, out_hbm, idx_v, rows_v, sem):
        base = _wid() * per
        pltpu.sync_copy(idx_hbm.at[pl.ds(base, per)], idx_v)
        pltpu.async_copy(table_hbm.at[idx_v], rows_v, sem).wait()
        pltpu.sync_copy(rows_v, out_hbm.at[pl.ds(base, per)])

    return k(table, idx)


def _gelu(z):
    return 0.5 * z * (1.0 + lax.erf(z * 0.7071067811865476))


_HI_MASK = -65536


def _pack_bf16_pair(a, b):
    ab = lax.bitcast_convert_type(
        a.astype(jnp.bfloat16).astype(jnp.float32), jnp.int32)
    bb = lax.bitcast_convert_type(
        b.astype(jnp.bfloat16).astype(jnp.float32), jnp.int32)
    return jnp.bitwise_or(
        jnp.bitwise_and(bb, jnp.full(bb.shape, _HI_MASK, jnp.int32)),
        lax.shift_right_logical(ab, 16))


def _unpack_lo(v):
    return lax.bitcast_convert_type(
        lax.shift_left(v, 16), jnp.float32).astype(jnp.bfloat16)


def _unpack_hi(v):
    return lax.bitcast_convert_type(
        jnp.bitwise_and(v, jnp.full(v.shape, _HI_MASK, jnp.int32)),
        jnp.float32).astype(jnp.bfloat16)


def _ln(z, g, beta):
    mu = jnp.mean(z, axis=-1, keepdims=True)
    d = z - mu
    var = jnp.mean(d * d, axis=-1, keepdims=True)
    return d * lax.rsqrt(var + 1e-5) * g + beta


def _fwd_body(xf_ref, u_ref, w1_ref, b1_ref, w2_ref, b2_ref, g_ref, beta_ref,
              gwh_ref, gwu_ref, gb_ref, h_ref, gi0_ref, gi1_ref, wc_ref):
    xf = xf_ref[...]
    h1 = _gelu(jnp.dot(xf, w1_ref[...], preferred_element_type=jnp.float32)
               + b1_ref[...])
    h2 = _gelu(jnp.dot(h1, w2_ref[...], preferred_element_type=jnp.float32)
               + b2_ref[...])
    h = _ln(h2, g_ref[...], beta_ref[...])
    h_ref[...] = _pack_bf16_pair(h[:, :128], h[:, 128:])
    u = u_ref[...]
    l = (jnp.dot(h, gwh_ref[...], preferred_element_type=jnp.float32)
         + jnp.dot(u, gwu_ref[...], preferred_element_type=jnp.float32)
         + gb_ref[...])
    BT = l.shape[0]
    iota = lax.broadcasted_iota(jnp.int32, (BT, E), 1)
    big = jnp.int32(999)
    m0 = jnp.max(l, axis=1, keepdims=True)
    i0 = jnp.min(jnp.where(l >= m0, iota, big), axis=1, keepdims=True)
    l2 = jnp.where(iota == i0, -1e30, l)
    m1 = jnp.max(l2, axis=1, keepdims=True)
    i1 = jnp.min(jnp.where(l2 >= m1, iota, big), axis=1, keepdims=True)
    w0 = 1.0 / (1.0 + jnp.exp(m1 - m0))
    w1 = 1.0 - w0
    vals = jnp.concatenate(
        [i0.astype(jnp.float32), i1.astype(jnp.float32)], axis=1)
    eye = (lax.broadcasted_iota(jnp.int32, (128, 128), 0)
           == lax.broadcasted_iota(jnp.int32, (128, 128), 1)
           ).astype(jnp.float32)
    r0 = []
    r1 = []
    for cch in range(BT // 128):
        blk = vals[cch * 128:(cch + 1) * 128, :2]
        ck = lax.dot_general(blk, eye, (((0,), (0,)), ((), ())),
                             preferred_element_type=jnp.float32)
        r0.append(ck[0:1])
        r1.append(ck[1:2])
    gi0_ref[...] = jnp.concatenate(r0, axis=0)
    gi1_ref[...] = jnp.concatenate(r1, axis=0)
    wc_ref[...] = jnp.concatenate(
        [jnp.broadcast_to(w0, (BT, 16)), jnp.broadcast_to(w1, (BT, 16)),
         jnp.zeros((BT, 96), jnp.float32)], axis=1)


def _fwd(xf, u, w1, b1, w2, b2, g, beta, gwh, gwu, gb):
    BT = 1024
    grid = B // BT
    nr = BT // 128
    return pl.pallas_call(
        _fwd_body,
        grid=(grid,),
        in_specs=[
            pl.BlockSpec((BT, IND), lambda i: (i, 0)),
            pl.BlockSpec((BT, 128), lambda i: (i, 0)),
            pl.BlockSpec((IND, HID), lambda i: (0, 0)),
            pl.BlockSpec((1, HID), lambda i: (0, 0)),
            pl.BlockSpec((HID, EMB), lambda i: (0, 0)),
            pl.BlockSpec((1, EMB), lambda i: (0, 0)),
            pl.BlockSpec((1, EMB), lambda i: (0, 0)),
            pl.BlockSpec((1, EMB), lambda i: (0, 0)),
            pl.BlockSpec((EMB, E), lambda i: (0, 0)),
            pl.BlockSpec((128, E), lambda i: (0, 0)),
            pl.BlockSpec((1, E), lambda i: (0, 0)),
        ],
        out_specs=[
            pl.BlockSpec((BT, 128), lambda i: (i, 0)),
            pl.BlockSpec((nr, 128), lambda i: (i, 0)),
            pl.BlockSpec((nr, 128), lambda i: (i, 0)),
            pl.BlockSpec((BT, 128), lambda i: (i, 0)),
        ],
        out_shape=[
            jax.ShapeDtypeStruct((B, 128), jnp.int32),
            jax.ShapeDtypeStruct((B // 128, 128), jnp.float32),
            jax.ShapeDtypeStruct((B // 128, 128), jnp.float32),
            jax.ShapeDtypeStruct((B, 128), jnp.float32),
        ],
        compiler_params=pltpu.CompilerParams(
            dimension_semantics=("parallel",)),
    )(xf, u, w1, b1, w2, b2, g, beta, gwh, gwu, gb)


def _route_body(gi0_ref, gi1_ref, pos_ref, te_ref):
    ep = jnp.concatenate([gi0_ref[...], gi1_ref[...]], axis=0).astype(jnp.int32)
    nrow = ep.shape[0]
    f32 = jnp.float32
    r128 = lax.broadcasted_iota(jnp.int32, (128, 128), 0)
    c128 = lax.broadcasted_iota(jnp.int32, (128, 128), 1)
    su128 = (r128 < c128).astype(f32)
    rr = lax.broadcasted_iota(jnp.int32, (nrow, nrow), 0)
    cc = lax.broadcasted_iota(jnp.int32, (nrow, nrow), 1)
    slr = (cc < rr).astype(f32)
    masks = []
    withins = []
    rowsums = []
    for e in range(E):
        me = (ep == e).astype(f32)
        masks.append(me)
        withins.append(jnp.dot(me, su128, preferred_element_type=f32))
        rowsums.append(jnp.sum(me, axis=1, keepdims=True))
    rs = jnp.concatenate(rowsums, axis=1)
    rowbase = jnp.dot(slr, rs, preferred_element_type=f32)
    ones_col = jnp.ones((nrow, 1), f32)
    counts = lax.dot_general(rs, ones_col, (((0,), (0,)), ((), ())),
                             preferred_element_type=f32)
    ci = counts.astype(jnp.int32)
    padded = ((ci + (T - 1)) // T) * T
    r16 = lax.broadcasted_iota(jnp.int32, (E, E), 0)
    c16 = lax.broadcasted_iota(jnp.int32, (E, E), 1)
    sl16 = (c16 < r16).astype(f32)
    qcol = jnp.dot(sl16, padded.astype(f32),
                   preferred_element_type=f32)
    acc = jnp.zeros_like(masks[0])
    for e in range(E):
        base_e = qcol[e:e + 1, 0:1] + rowbase[:, e:e + 1]
        acc = acc + masks[e] * (base_e + withins[e])
    pos_ref[...] = acc.astype(jnp.int32)
    jt = lax.broadcasted_iota(jnp.int32, (E, NT), 1) * T
    cmp = (qcol.astype(jnp.int32) <= jt).astype(jnp.int32)
    te = jnp.sum(cmp, axis=0, keepdims=True) - 1
    te_ref[...] = jnp.broadcast_to(te, (8, NT))


def _route(gi0, gi1):
    nrow = NPAIR // 128
    return pl.pallas_call(
        _route_body,
        in_specs=[
            pl.BlockSpec((B // 128, 128), lambda: (0, 0)),
            pl.BlockSpec((B // 128, 128), lambda: (0, 0)),
        ],
        out_specs=[
            pl.BlockSpec((nrow, 128), lambda: (0, 0)),
            pl.BlockSpec((8, NT), lambda: (0, 0)),
        ],
        out_shape=[
            jax.ShapeDtypeStruct((nrow, 128), jnp.int32),
            jax.ShapeDtypeStruct((8, NT), jnp.int32),
        ],
    )(gi0, gi1)


def _sc_dispatch(h, pos):
    per = NPAIR // NW
    SUB = 128
    nsub = per // SUB

    @functools.partial(
        pl.kernel,
        out_type=jax.ShapeDtypeStruct((P, 128), jnp.int32),
        mesh=plsc.VectorSubcoreMesh(**_SC_MESH),
        scratch_types=[
            pltpu.VMEM((SUB,), jnp.int32),
            pltpu.VMEM((SUB,), jnp.int32),
            pltpu.VMEM((SUB,), jnp.int32),
            pltpu.VMEM((SUB,), jnp.int32),
            pltpu.VMEM((SUB, 128), jnp.int32),
            pltpu.VMEM((SUB, 128), jnp.int32),
            pltpu.SemaphoreType.DMA,
            pltpu.SemaphoreType.DMA,
            pltpu.SemaphoreType.DMA,
            pltpu.SemaphoreType.DMA,
        ],
    )
    def k(h_hbm, pos_hbm, hs_hbm, tok0, tok1, pos0, pos1, r0, r1,
          gs0, gs1, ss0, ss1):
        toks = [tok0, tok1]
        poss = [pos0, pos1]
        rows = [r0, r1]
        gss = [gs0, gs1]
        sss = [ss0, ss1]
        gh = [None, None]
        sh = [None, None]
        w = _wid()
        for s in range(nsub):
            b = s % 2
            if s >= 2:
                sh[b].wait()
            base = w * per + s * SUB
            bm = base % B

            def fill(c, _):
                toks[b][pl.ds(c * 16, 16)] = (
                    bm + c * 16 + lax.iota(jnp.int32, 16))
                return 0

            lax.fori_loop(0, SUB // 16, fill, 0)
            pltpu.sync_copy(pos_hbm.at[base // 128], poss[b])
            gh[b] = pltpu.async_copy(h_hbm.at[toks[b]], rows[b], gss[b])
            if s >= 1:
                c = (s - 1) % 2
                gh[c].wait()
                sh[c] = pltpu.async_copy(rows[c], hs_hbm.at[poss[c]], sss[c])
        last = (nsub - 1) % 2
        gh[last].wait()
        sh[last] = pltpu.async_copy(rows[last], hs_hbm.at[poss[last]],
                                    sss[last])
        if nsub >= 2:
            sh[(nsub - 2) % 2].wait()
        sh[last].wait()

    return k(h, pos)


def _expert_body(te_ref, hs_ref, w1_ref, b1_ref, g_ref, beta_ref,
                 w2_ref, b2_ref, ys_ref):
    hs32 = hs_ref[...]
    w1 = w1_ref[0]
    z = (jnp.dot(_unpack_lo(hs32), w1[:128],
                 preferred_element_type=jnp.float32)
         + jnp.dot(_unpack_hi(hs32), w1[128:],
                   preferred_element_type=jnp.float32)) + b1_ref[0]
    z = _gelu(z)
    z = _ln(z, g_ref[0], beta_ref[0])
    y = jnp.dot(z.astype(jnp.bfloat16), w2_ref[0],
                preferred_element_type=jnp.float32) + b2_ref[0]
    ys_ref[...] = y


def _experts(te, hs, w1, b1, g, beta, w2p, b2p):
    grid_spec = pltpu.PrefetchScalarGridSpec(
        num_scalar_prefetch=1,
        grid=(NT,),
        in_specs=[
            pl.BlockSpec((T, 128), lambda i, te: (i, 0)),
            pl.BlockSpec((1, EMB, EMB), lambda i, te: (te[i], 0, 0)),
            pl.BlockSpec((1, 1, EMB), lambda i, te: (te[i], 0, 0)),
            pl.BlockSpec((1, 1, EMB), lambda i, te: (te[i], 0, 0)),
            pl.BlockSpec((1, 1, EMB), lambda i, te: (te[i], 0, 0)),
            pl.BlockSpec((1, EMB, CP), lambda i, te: (te[i], 0, 0)),
            pl.BlockSpec((1, 1, CP), lambda i, te: (te[i], 0, 0)),
        ],
        out_specs=pl.BlockSpec((T, CP), lambda i, te: (i, 0)),
    )
    return pl.pallas_call(
        _expert_body,
        grid_spec=grid_spec,
        out_shape=jax.ShapeDtypeStruct((P, CP), jnp.float32),
        compiler_params=pltpu.CompilerParams(
            dimension_semantics=("arbitrary",)),
    )(te, hs, w1, b1, g, beta, w2p, b2p)


def _sc_combine(ys, pos2d, wc):
    per = B // NW
    SUB = 128
    nsub = per // SUB

    @functools.partial(
        pl.kernel,
        out_type=jax.ShapeDtypeStruct((B, C), jnp.float32),
        mesh=plsc.VectorSubcoreMesh(**_SC_MESH),
        scratch_types=[
            pltpu.VMEM((SUB,), jnp.int32),
            pltpu.VMEM((SUB,), jnp.int32),
            pltpu.VMEM((SUB,), jnp.int32),
            pltpu.VMEM((SUB,), jnp.int32),
            pltpu.VMEM((SUB, 128), jnp.float32),
            pltpu.VMEM((SUB, 128), jnp.float32),
            pltpu.VMEM((SUB, CP), jnp.float32),
            pltpu.VMEM((SUB, CP), jnp.float32),
            pltpu.VMEM((SUB, CP), jnp.float32),
            pltpu.VMEM((SUB, CP), jnp.float32),
            pltpu.VMEM((SUB, C), jnp.float32),
            pltpu.SemaphoreType.DMA,
            pltpu.SemaphoreType.DMA,
            pltpu.SemaphoreType.DMA,
            pltpu.SemaphoreType.DMA,
        ],
    )
    def k(ys_hbm, pos_hbm, wc_hbm, out_hbm,
          i0a, i0b, i1a, i1b, wca, wcb, g0a, g0b, g1a, g1b, ob,
          s0a, s0b, s1a, s1b):
        i0s = [i0a, i0b]
        i1s = [i1a, i1b]
        wcs = [wca, wcb]
        g0s = [g0a, g0b]
        g1s = [g1a, g1b]
        sm0 = [s0a, s0b]
        sm1 = [s1a, s1b]
        h0 = [None, None]
        h1 = [None, None]
        w = _wid()

        def start(s):
            b = s % 2
            base = w * per + s * SUB
            row = base // 128
            pltpu.sync_copy(pos_hbm.at[row], i0s[b])
            pltpu.sync_copy(pos_hbm.at[(B // 128) + row], i1s[b])
            pltpu.sync_copy(wc_hbm.at[pl.ds(base, SUB)], wcs[b])
            h0[b] = pltpu.async_copy(ys_hbm.at[i0s[b]], g0s[b], sm0[b])
            h1[b] = pltpu.async_copy(ys_hbm.at[i1s[b]], g1s[b], sm1[b])

        start(0)
        for s in range(nsub):
            b = s % 2
            if s + 1 < nsub:
                start(s + 1)
            h0[b].wait()
            h1[b].wait()
            g0 = g0s[b]
            g1 = g1s[b]
            wc_v = wcs[b]

            def body(r, _):
                a = wc_v[r, pl.ds(0, 16)]
                c = wc_v[r, pl.ds(16, 16)]
                for off in (0, 16, 32, 48, 64, 80, C - 16):
                    sl = pl.ds(off, 16)
                    ob[r, sl] = a * g0[r, sl] + c * g1[r, sl]
                return 0

            lax.fori_loop(0, SUB, body, 0)
            base = w * per + s * SUB
            pltpu.sync_copy(ob, out_hbm.at[pl.ds(base, SUB)])

    return k(ys, pos2d, wc)


def kernel(x, user_ids, bb_W1, bb_b1, bb_W2, bb_b2, bb_g, bb_beta,
           user_table, gate_W, gate_b,
           ex_W1, ex_b1, ex_g, ex_beta, ex_W2, ex_b2):
    i32 = jnp.int32
    xf = x.reshape(B, IND)
    table_p = jnp.pad(user_table, ((0, 0), (0, 128 - UDIM)))
    u = _sc_gather_u(table_p, user_ids.astype(i32))
    gwu_p = jnp.pad(gate_W[EMB:], ((0, 128 - UDIM), (0, 0)))
    h, gi0, gi1, wc = _fwd(xf, u, bb_W1, bb_b1.reshape(1, HID), bb_W2,
                           bb_b2.reshape(1, EMB), bb_g.reshape(1, EMB),
                           bb_beta.reshape(1, EMB), gate_W[:EMB], gwu_p,
                           gate_b.reshape(1, E))
    pos2d, te8 = _route(gi0, gi1)
    te = te8[0]
    hs = _sc_dispatch(h, pos2d)
    w2p = jnp.pad(ex_W2, ((0, 0), (0, 0), (0, CP - C))).astype(jnp.bfloat16)
    b2p = jnp.pad(ex_b2, ((0, 0), (0, CP - C))).reshape(E, 1, CP)
    ys = _experts(te, hs, ex_W1.astype(jnp.bfloat16), ex_b1.reshape(E, 1, EMB),
                  ex_g.reshape(E, 1, EMB), ex_beta.reshape(E, 1, EMB),
                  w2p, b2p)
    return _sc_combine(ys, pos2d, wc)
